# baseline (speedup 1.0000x reference)
import functools

import jax
import jax.numpy as jnp
from jax import lax
from jax.experimental import pallas as pl
from jax.experimental.pallas import tpu as pltpu
from jax.experimental.pallas import tpu_sc as plsc

NW = 32
TPW = 64
RPW = 32
NTAB = 9
SL_OFF = (0, 2048, 4096, 6144, 8192)
SL_W = (2048, 2048, 2048, 2048, 1792)
NSLICE = len(SL_OFF)
CSL = 2048
DMAIN = 9984


def _level_idx(x, low, high, n):
    xc = jnp.clip(x, low, high)
    idx = jnp.round((xc - low) / (high - low) * (n - 1)).astype(jnp.int32)
    return jnp.clip(idx, 0, n - 1)


def _all_indices(input, n_lvl, n_time):
    SIGNAL_MIN, SIGNAL_MAX = -5.0, 5.0
    MAG_MIN, MAG_MAX = -10.0, 10.0
    ENERGY_MIN, ENERGY_MAX = -10.0, 10.0

    t = input[:, 0] - input[0, 0]
    xyz = input[:, 1:]

    idx_x = _level_idx(jnp.clip(xyz[:, 0], SIGNAL_MIN, SIGNAL_MAX), SIGNAL_MIN, SIGNAL_MAX, n_lvl)
    idx_y = _level_idx(jnp.clip(xyz[:, 1], SIGNAL_MIN, SIGNAL_MAX), SIGNAL_MIN, SIGNAL_MAX, n_lvl)
    idx_z = _level_idx(jnp.clip(xyz[:, 2], SIGNAL_MIN, SIGNAL_MAX), SIGNAL_MIN, SIGNAL_MAX, n_lvl)

    mags = jnp.sqrt(jnp.sum(jnp.square(xyz), axis=1))
    idx_mag = _level_idx(mags, MAG_MIN, MAG_MAX, n_lvl)

    dt = t[1:] - t[:-1]
    jerk_body = (xyz[1:] - xyz[:-1]) / dt[:, None]
    jerk = jnp.concatenate([jnp.zeros((1, 3), dtype=input.dtype), jerk_body], axis=0)

    idx_xj = _level_idx(jnp.clip(jerk[:, 0], SIGNAL_MIN, SIGNAL_MAX), SIGNAL_MIN, SIGNAL_MAX, n_lvl)
    idx_yj = _level_idx(jnp.clip(jerk[:, 1], SIGNAL_MIN, SIGNAL_MAX), SIGNAL_MIN, SIGNAL_MAX, n_lvl)
    idx_zj = _level_idx(jnp.clip(jerk[:, 2], SIGNAL_MIN, SIGNAL_MAX), SIGNAL_MIN, SIGNAL_MAX, n_lvl)

    jerk_mags = jnp.sqrt(jnp.sum(jnp.square(jerk), axis=1))
    idx_magj = _level_idx(jerk_mags, MAG_MIN, MAG_MAX, n_lvl)

    idx_time = _level_idx(t, 0.0, float(n_time), n_time)

    energy = jnp.sum(jnp.square(xyz), axis=0) / xyz.shape[0]
    T = input.shape[0]
    idx_ex = jnp.full((T,), _level_idx(energy[0], ENERGY_MIN, ENERGY_MAX, n_lvl), jnp.int32)
    idx_ey = jnp.full((T,), _level_idx(energy[1], ENERGY_MIN, ENERGY_MAX, n_lvl), jnp.int32)
    idx_ez = jnp.full((T,), _level_idx(energy[2], ENERGY_MIN, ENERGY_MAX, n_lvl), jnp.int32)

    return jnp.stack([idx_x, idx_y, idx_z, idx_mag, idx_xj, idx_yj, idx_zj,
                      idx_magj, idx_time, idx_ex, idx_ey, idx_ez], axis=0)


def _sc_gather_product(idx5, tabs_v):
    mesh = plsc.VectorSubcoreMesh(core_axis_name="c", subcore_axis_name="s")

    @functools.partial(
        pl.kernel,
        mesh=mesh,
        out_type=jax.ShapeDtypeStruct((NW, NSLICE, CSL), jnp.float32),
        compiler_params=pltpu.CompilerParams(use_tc_tiling_on_sc=True),
        scratch_types=[
            pltpu.VMEM((NTAB, RPW, 2), jnp.int32),
            pltpu.VMEM((2, NTAB, 2, CSL), jnp.float32),
            pltpu.VMEM((NSLICE, CSL), jnp.float32),
            pltpu.SemaphoreType.DMA,
            pltpu.SemaphoreType.DMA,
        ],
    )
    def sc_k(idx_hbm, t0, t1, t2, t3, t4, t5, t6, t7, t8, out_hbm,
             idx_v, buf, acc, sem0, sem1):
        tabs = (t0, t1, t2, t3, t4, t5, t6, t7, t8)
        sems = (sem0, sem1)
        wid = lax.axis_index("s") * 2 + lax.axis_index("c")

        pltpu.sync_copy(idx_hbm.at[wid], idx_v)

        for s in range(NSLICE):
            off, w = SL_OFF[s], SL_W[s]

            def zero_g(g, carry):
                acc[s, pl.ds(g * 16, 16)] = jnp.zeros((16,), jnp.float32)
                return carry

            lax.fori_loop(0, w // 16, zero_g, 0)

            def src(k, r):
                return tabs[k].at[idx_v.at[k, r], pl.ds(off, w)]

            def dst(p, k):
                return buf.at[p, k, :, pl.ds(0, w)]

            for p in (0, 1):
                for k in range(NTAB):
                    pltpu.async_copy(src(k, p), dst(p, k), sems[p])

            def round_pair(i2, carry):
                for p in (0, 1):
                    r = i2 * 2 + p
                    for k in range(NTAB):
                        pltpu.make_async_copy(src(k, r), dst(p, k),
                                              sems[p]).wait()

                    def grp(g, c2):
                        col = pl.ds(g * 16, 16)
                        v0 = buf[p, 0, 0, col]
                        v1 = buf[p, 0, 1, col]
                        for k in range(1, NTAB):
                            v0 = v0 * buf[p, k, 0, col]
                            v1 = v1 * buf[p, k, 1, col]
                        acc[s, col] = acc[s, col] + (v0 + v1)
                        return c2

                    lax.fori_loop(0, w // 16, grp, 0)

                    rn = r + 2

                    @pl.when(rn < RPW)
                    def _():
                        for k in range(NTAB):
                            pltpu.async_copy(src(k, rn), dst(p, k), sems[p])
                return carry

            lax.fori_loop(0, RPW // 2, round_pair, 0)

        pltpu.sync_copy(acc, out_hbm.at[wid])

    return sc_k(idx5, *tabs_v)


def _finale_body(parts, tail, erows, out_ref):
    s = jnp.sum(parts[...], axis=0, keepdims=True)
    full = jnp.concatenate([s[:, :DMAIN], tail[:, :16]], axis=1)
    e = erows[0:1] * erows[1:2] * erows[2:3]
    out_ref[...] = jnp.tanh(full * e)


def _tc_finale(partials, tail, e_rows, D):
    PW = NSLICE * CSL
    out = pl.pallas_call(
        _finale_body,
        grid=(1,),
        in_specs=[pl.BlockSpec((NW, PW), lambda i: (0, 0)),
                  pl.BlockSpec((1, 128), lambda i: (0, 0)),
                  pl.BlockSpec((3, D), lambda i: (0, 0))],
        out_specs=pl.BlockSpec((1, D), lambda i: (0, 0)),
        out_shape=jax.ShapeDtypeStruct((1, D), jnp.float32),
    )(partials, tail, e_rows)
    return out[0]


def _tail_block_body(*refs):
    ins = refs[:NTAB]
    out_ref = refs[NTAB]
    for k in range(NTAB):
        out_ref[k, :ins[k].shape[0]] = ins[k][...]


def _tc_tail_blocks(tabs, nt):
    blk = DMAIN // 128
    out = pl.pallas_call(
        _tail_block_body,
        grid=(1,),
        in_specs=[pl.BlockSpec((t.shape[0], 128), lambda i, b=blk: (0, b))
                  for t in tabs],
        out_specs=pl.BlockSpec((NTAB, nt, 128), lambda i: (0, 0, 0)),
        out_shape=jax.ShapeDtypeStruct((NTAB, nt, 128), jnp.float32),
    )(*tabs)
    return out


def kernel(input, T_x, T_y, T_z, T_mag, T_xj, T_yj, T_zj, T_magj, T_ex, T_ey, T_ez, T_time):
    n_lvl = T_x.shape[0]
    n_time = T_time.shape[0]
    D = T_x.shape[1]
    T = input.shape[0]

    idx_all = _all_indices(input, n_lvl, n_time)

    idx9 = idx_all[:NTAB].reshape(NTAB, NW, RPW, 2)
    idx9 = jnp.transpose(idx9, (1, 0, 2, 3)).astype(jnp.int32)

    tabs = (T_x, T_y, T_z, T_mag, T_xj, T_yj, T_zj, T_magj, T_time)
    partials = _sc_gather_product(idx9, tabs).reshape(NW, NSLICE * CSL)

    tail_cat = _tc_tail_blocks(tabs, n_time)
    tail16 = jnp.take(tail_cat[0], idx_all[0], axis=0)[:, :16]
    for k in range(1, NTAB):
        tail16 = tail16 * jnp.take(tail_cat[k], idx_all[k], axis=0)[:, :16]
    tail = jnp.zeros((1, 128), jnp.float32).at[0, :16].set(jnp.sum(tail16, axis=0))

    e_rows = jnp.concatenate(
        [jnp.take(t, idx_all[NTAB + j, 0][None], axis=0)
         for j, t in enumerate((T_ex, T_ey, T_ez))], axis=0)

    out = _tc_finale(partials, tail, e_rows, D)
    return out


# speedup vs baseline: 1.5713x; 1.5713x over previous
import functools

import jax
import jax.numpy as jnp
from jax import lax
from jax.experimental import pallas as pl
from jax.experimental.pallas import tpu as pltpu
from jax.experimental.pallas import tpu_sc as plsc

NW = 32
TPW = 64
RPW = 32
NTAB = 9
NBIT = 32
WSTR = 384
DPACK = NBIT * WSTR


def _level_idx(x, low, high, n):
    xc = jnp.clip(x, low, high)
    idx = jnp.round((xc - low) / (high - low) * (n - 1)).astype(jnp.int32)
    return jnp.clip(idx, 0, n - 1)


def _all_indices(input, n_lvl, n_time):
    SIGNAL_MIN, SIGNAL_MAX = -5.0, 5.0
    MAG_MIN, MAG_MAX = -10.0, 10.0
    ENERGY_MIN, ENERGY_MAX = -10.0, 10.0

    t = input[:, 0] - input[0, 0]
    xyz = input[:, 1:]

    idx_x = _level_idx(jnp.clip(xyz[:, 0], SIGNAL_MIN, SIGNAL_MAX), SIGNAL_MIN, SIGNAL_MAX, n_lvl)
    idx_y = _level_idx(jnp.clip(xyz[:, 1], SIGNAL_MIN, SIGNAL_MAX), SIGNAL_MIN, SIGNAL_MAX, n_lvl)
    idx_z = _level_idx(jnp.clip(xyz[:, 2], SIGNAL_MIN, SIGNAL_MAX), SIGNAL_MIN, SIGNAL_MAX, n_lvl)

    mags = jnp.sqrt(jnp.sum(jnp.square(xyz), axis=1))
    idx_mag = _level_idx(mags, MAG_MIN, MAG_MAX, n_lvl)

    dt = t[1:] - t[:-1]
    jerk_body = (xyz[1:] - xyz[:-1]) / dt[:, None]
    jerk = jnp.concatenate([jnp.zeros((1, 3), dtype=input.dtype), jerk_body], axis=0)

    idx_xj = _level_idx(jnp.clip(jerk[:, 0], SIGNAL_MIN, SIGNAL_MAX), SIGNAL_MIN, SIGNAL_MAX, n_lvl)
    idx_yj = _level_idx(jnp.clip(jerk[:, 1], SIGNAL_MIN, SIGNAL_MAX), SIGNAL_MIN, SIGNAL_MAX, n_lvl)
    idx_zj = _level_idx(jnp.clip(jerk[:, 2], SIGNAL_MIN, SIGNAL_MAX), SIGNAL_MIN, SIGNAL_MAX, n_lvl)

    jerk_mags = jnp.sqrt(jnp.sum(jnp.square(jerk), axis=1))
    idx_magj = _level_idx(jerk_mags, MAG_MIN, MAG_MAX, n_lvl)

    idx_time = _level_idx(t, 0.0, float(n_time), n_time)

    energy = jnp.sum(jnp.square(xyz), axis=0) / xyz.shape[0]
    T = input.shape[0]
    idx_ex = jnp.full((T,), _level_idx(energy[0], ENERGY_MIN, ENERGY_MAX, n_lvl), jnp.int32)
    idx_ey = jnp.full((T,), _level_idx(energy[1], ENERGY_MIN, ENERGY_MAX, n_lvl), jnp.int32)
    idx_ez = jnp.full((T,), _level_idx(energy[2], ENERGY_MIN, ENERGY_MAX, n_lvl), jnp.int32)

    return jnp.stack([idx_x, idx_y, idx_z, idx_mag, idx_xj, idx_yj, idx_zj,
                      idx_magj, idx_time, idx_ex, idx_ey, idx_ez], axis=0)


def _pack_body(in_ref, out_ref):
    x = in_ref[...]
    rows, d = x.shape
    pad = jnp.ones((rows, DPACK - d), x.dtype)
    xp = jnp.concatenate([x, pad], axis=1)
    w = jnp.zeros((rows, WSTR), jnp.int32)
    for j in range(NBIT):
        bit = (xp[:, WSTR * j:WSTR * (j + 1)] < 0).astype(jnp.int32)
        w = w | (bit << j)
    out_ref[...] = w


def _tc_pack(table):
    R, D = table.shape
    RB = 64
    return pl.pallas_call(
        _pack_body,
        grid=(R // RB,),
        in_specs=[pl.BlockSpec((RB, D), lambda i: (i, 0))],
        out_specs=pl.BlockSpec((RB, WSTR), lambda i: (i, 0)),
        out_shape=jax.ShapeDtypeStruct((R, WSTR), jnp.int32),
    )(table)


def _sc_gather_product(idx9, tabs_w):
    mesh = plsc.VectorSubcoreMesh(core_axis_name="c", subcore_axis_name="s")

    @functools.partial(
        pl.kernel,
        mesh=mesh,
        out_type=jax.ShapeDtypeStruct((NW, NBIT, WSTR), jnp.int32),
        compiler_params=pltpu.CompilerParams(use_tc_tiling_on_sc=True),
        scratch_types=[
            pltpu.VMEM((NTAB, RPW, 2), jnp.int32),
            pltpu.VMEM((2, NTAB, 2, WSTR), jnp.int32),
            pltpu.VMEM((NBIT, WSTR), jnp.int32),
            pltpu.SemaphoreType.DMA,
            pltpu.SemaphoreType.DMA,
        ],
    )
    def sc_k(idx_hbm, t0, t1, t2, t3, t4, t5, t6, t7, t8, out_hbm,
             idx_v, buf, acc, sem0, sem1):
        tabs = (t0, t1, t2, t3, t4, t5, t6, t7, t8)
        sems = (sem0, sem1)
        wid = lax.axis_index("s") * 2 + lax.axis_index("c")

        pltpu.sync_copy(idx_hbm.at[wid], idx_v)

        def zero_g(g, carry):
            acc[0, pl.ds(g * 16, 16)] = jnp.zeros((16,), jnp.int32)
            return carry

        def zero_j(j, carry):
            def zg(g, c):
                acc[j, pl.ds(g * 16, 16)] = jnp.zeros((16,), jnp.int32)
                return c
            return lax.fori_loop(0, WSTR // 16, zg, carry)

        lax.fori_loop(0, NBIT, zero_j, 0)

        def src(k, r):
            return tabs[k].at[idx_v.at[k, r]]

        def dst(p, k):
            return buf.at[p, k]

        for p in (0, 1):
            for k in range(NTAB):
                pltpu.async_copy(src(k, p), dst(p, k), sems[p])

        def round_pair(i2, carry):
            for p in (0, 1):
                r = i2 * 2 + p
                for k in range(NTAB):
                    pltpu.make_async_copy(src(k, r), dst(p, k),
                                          sems[p]).wait()

                def grp(g, c2):
                    col = pl.ds(g * 16, 16)
                    w0 = buf[p, 0, 0, col]
                    w1 = buf[p, 0, 1, col]
                    for k in range(1, NTAB):
                        w0 = w0 ^ buf[p, k, 0, col]
                        w1 = w1 ^ buf[p, k, 1, col]
                    one = jnp.ones((16,), jnp.int32)
                    for j in range(NBIT):
                        b = ((w0 >> j) & one) + ((w1 >> j) & one)
                        acc[j, col] = acc[j, col] + b
                    return c2

                lax.fori_loop(0, WSTR // 16, grp, 0)

                rn = r + 2

                @pl.when(rn < RPW)
                def _():
                    for k in range(NTAB):
                        pltpu.async_copy(src(k, rn), dst(p, k), sems[p])
            return carry

        lax.fori_loop(0, RPW // 2, round_pair, 0)

        pltpu.sync_copy(acc, out_hbm.at[wid])

    return sc_k(idx9, *tabs_w)


def _finale_body(parts, erows, out_ref):
    cnt = jnp.sum(parts[...], axis=0, keepdims=True)
    T = 2048.0
    s = T - 2.0 * cnt.astype(jnp.float32)
    d = out_ref.shape[1]
    e = erows[0:1] * erows[1:2] * erows[2:3]
    out_ref[...] = jnp.tanh(s[:, :d] * e)


def _tc_finale(partials, e_rows, D):
    out = pl.pallas_call(
        _finale_body,
        grid=(1,),
        in_specs=[pl.BlockSpec((NW, DPACK), lambda i: (0, 0)),
                  pl.BlockSpec((3, D), lambda i: (0, 0))],
        out_specs=pl.BlockSpec((1, D), lambda i: (0, 0)),
        out_shape=jax.ShapeDtypeStruct((1, D), jnp.float32),
    )(partials, e_rows)
    return out[0]


def kernel(input, T_x, T_y, T_z, T_mag, T_xj, T_yj, T_zj, T_magj, T_ex, T_ey, T_ez, T_time):
    n_lvl = T_x.shape[0]
    n_time = T_time.shape[0]
    D = T_x.shape[1]

    idx_all = _all_indices(input, n_lvl, n_time)

    idx9 = idx_all[:NTAB].reshape(NTAB, NW, RPW, 2)
    idx9 = jnp.transpose(idx9, (1, 0, 2, 3)).astype(jnp.int32)

    tabs = (T_x, T_y, T_z, T_mag, T_xj, T_yj, T_zj, T_magj, T_time)
    tabs_w = tuple(_tc_pack(t) for t in tabs)

    partials = _sc_gather_product(idx9, tabs_w).reshape(NW, DPACK)

    e_rows = jnp.concatenate(
        [jnp.take(t, idx_all[NTAB + j, 0][None], axis=0)
         for j, t in enumerate((T_ex, T_ey, T_ez))], axis=0)

    out = _tc_finale(partials, e_rows, D)
    return out
